# baseline (speedup 1.0000x reference)
import functools

import numpy as np

import jax
import jax.numpy as jnp
from jax import lax
from jax.experimental import pallas as pl
from jax.experimental.pallas import tpu as pltpu
from jax.experimental.pallas import tpu_sc as plsc

_BR = 2048
_L = 16384
_E = 128
_O = 8
_NBLK = _L // _BR


def _pos_tables(seq_len, emb_size, br):
    pos = np.arange(1, emb_size + 1, dtype=np.float64)
    w = 1.0 / np.power(10000.0, 2.0 * pos / emb_size)
    t = np.arange(1, br + 1, dtype=np.float64).reshape(-1, 1)
    a0 = t * w
    even = (np.arange(emb_size) % 2) == 0
    p0 = np.where(even, np.sin(a0), np.cos(a0)).astype(np.float32)
    q0 = np.where(even, np.cos(a0), -np.sin(a0)).astype(np.float32)
    nblk = seq_len // br
    off = (br * np.arange(nblk, dtype=np.float64).reshape(-1, 1)) * w
    cbx = np.repeat(np.cos(off), _O, axis=0).astype(np.float32)
    sbx = np.repeat(np.sin(off), _O, axis=0).astype(np.float32)
    return (np.ascontiguousarray(p0.T), np.ascontiguousarray(q0.T),
            cbx, sbx)


_P0T, _Q0T, _CBX, _SBX = _pos_tables(_L, _E, _BR)


def _sc_gather(idx, table, start, num):
    info = plsc.get_sparse_core_info()
    nw = info.num_cores * info.num_subcores
    d = table.shape[1]
    b_per_w = num // nw
    ch = 128
    n_ch = b_per_w // ch

    mesh = plsc.VectorSubcoreMesh(core_axis_name="c", subcore_axis_name="s")

    @functools.partial(
        pl.kernel,
        mesh=mesh,
        out_type=jax.ShapeDtypeStruct((num, d), jnp.float32),
        scratch_types=[
            pltpu.VMEM((b_per_w,), jnp.int32),
            pltpu.VMEM((b_per_w, d), jnp.float32),
            pltpu.SemaphoreType.DMA,
            pltpu.SemaphoreType.DMA,
        ],
    )
    def gather_kernel(idx_hbm, table_hbm, out_hbm, idx_v, rows_v, sem,
                      wsem):
        wid = lax.axis_index("s") * info.num_cores + lax.axis_index("c")
        base = wid * b_per_w
        pltpu.sync_copy(idx_hbm.at[pl.ds(start + base, b_per_w)], idx_v)
        copies = [
            pltpu.async_copy(
                table_hbm.at[idx_v.at[pl.ds(j * ch, ch)]],
                rows_v.at[pl.ds(j * ch, ch)],
                sem,
            )
            for j in range(n_ch)
        ]
        writes = []
        for j, c in enumerate(copies):
            c.wait()
            writes.append(pltpu.async_copy(
                rows_v.at[pl.ds(j * ch, ch)],
                out_hbm.at[pl.ds(base + j * ch, ch)],
                wsem,
            ))
        for w in writes:
            w.wait()

    return gather_kernel(idx, table)


def _tc_pos_proj(wq, bq):
    o, e = wq.shape
    br = _BR
    nblk = _NBLK

    def body(p0t_ref, q0t_ref, cbx_ref, sbx_ref, wq_ref, bq_ref, out_ref):
        wqt = jnp.concatenate([wq_ref[...]] * nblk, axis=0)
        bqt = jnp.concatenate([bq_ref[...]] * nblk, axis=0)
        a = lax.dot_general(
            wqt * cbx_ref[...], p0t_ref[...], (((1,), (0,)), ((), ())),
            preferred_element_type=jnp.float32)
        b = lax.dot_general(
            wqt * sbx_ref[...], q0t_ref[...], (((1,), (0,)), ((), ())),
            preferred_element_type=jnp.float32)
        out_ref[...] = (a + b + bqt).reshape(nblk, o, br)

    return pl.pallas_call(
        body,
        in_specs=[
            pl.BlockSpec((e, br), lambda: (0, 0)),
            pl.BlockSpec((e, br), lambda: (0, 0)),
            pl.BlockSpec((nblk * o, e), lambda: (0, 0)),
            pl.BlockSpec((nblk * o, e), lambda: (0, 0)),
            pl.BlockSpec((o, e), lambda: (0, 0)),
            pl.BlockSpec((o, 1), lambda: (0, 0)),
        ],
        out_specs=pl.BlockSpec((nblk, o, br), lambda: (0, 0, 0)),
        out_shape=jax.ShapeDtypeStruct((nblk, o, br), jnp.float32),
    )(jnp.asarray(_P0T), jnp.asarray(_Q0T), jnp.asarray(_CBX),
      jnp.asarray(_SBX), wq, bq.reshape(o, 1))


def _tc_project(emb, wq, pp, blk0, carry=None):
    o = wq.shape[0]
    e = wq.shape[1]
    br = _BR
    nb = emb.shape[0] // br

    def body(*refs):
        if carry is None:
            emba_ref, embb_ref, wq_ref, ppa_ref, ppb_ref, out_ref = refs
        else:
            _, emba_ref, embb_ref, wq_ref, ppa_ref, ppb_ref, out_ref = refs
        wq2 = wq_ref[...]
        z = jnp.zeros_like(wq2)
        c0 = jnp.concatenate([wq2, z], axis=1)
        c1 = jnp.concatenate([z, wq2], axis=1)
        w2 = jnp.concatenate(
            [c0.reshape(o, 1, 2 * e), c1.reshape(o, 1, 2 * e)], axis=1
        ).reshape(2 * o, 2 * e)
        emb2 = jnp.concatenate([emba_ref[...], embb_ref[...]], axis=1)
        acc = lax.dot_general(
            w2, emb2, (((1,), (1,)), ((), ())),
            preferred_element_type=jnp.float32)
        pp2 = jnp.concatenate(
            [ppa_ref[...].reshape(o, 1, br), ppb_ref[...].reshape(o, 1, br)],
            axis=1).reshape(o, 2 * br)
        out_ref[...] = acc.reshape(o, 2 * br) + pp2

    in_specs = [
        pl.BlockSpec((br, e), lambda j: (2 * j, 0)),
        pl.BlockSpec((br, e), lambda j: (2 * j + 1, 0)),
        pl.BlockSpec((o, e), lambda j: (0, 0)),
        pl.BlockSpec((1, o, br), lambda j: (blk0 + 2 * j, 0, 0)),
        pl.BlockSpec((1, o, br), lambda j: (blk0 + 2 * j + 1, 0, 0)),
    ]
    args = [emb, emb, wq, pp, pp]
    kwargs = {}
    if carry is not None:
        in_specs = [pl.BlockSpec(memory_space=pl.ANY)] + in_specs
        args = [carry] + args
        kwargs["input_output_aliases"] = {0: 0}

    return pl.pallas_call(
        body,
        grid=(nb // 2,),
        in_specs=in_specs,
        out_specs=pl.BlockSpec((o, 2 * br), lambda j: (0, blk0 // 2 + j)),
        out_shape=jax.ShapeDtypeStruct((o, _L), jnp.float32),
        **kwargs,
    )(*args)


def kernel(inputs, table, Wq, bq):
    pp = _tc_pos_proj(Wq, bq)
    l = inputs.shape[0]
    na = 6 * _BR
    emb_a = _sc_gather(inputs, table, 0, na)
    emb_b = _sc_gather(inputs, table, na, l - na)
    out_t = _tc_project(emb_a, Wq, pp, blk0=0)
    out_t = _tc_project(emb_b, Wq, pp, blk0=6, carry=out_t)
    return out_t.T


# speedup vs baseline: 1.0144x; 1.0144x over previous
import functools

import numpy as np

import jax
import jax.numpy as jnp
from jax import lax
from jax.experimental import pallas as pl
from jax.experimental.pallas import tpu as pltpu
from jax.experimental.pallas import tpu_sc as plsc

_BR = 1024
_L = 16384
_E = 128
_O = 8
_NBLK = _L // _BR


def _pos_tables(seq_len, emb_size, br):
    pos = np.arange(1, emb_size + 1, dtype=np.float64)
    w = 1.0 / np.power(10000.0, 2.0 * pos / emb_size)
    t = np.arange(1, br + 1, dtype=np.float64).reshape(-1, 1)
    a0 = t * w
    even = (np.arange(emb_size) % 2) == 0
    p0 = np.where(even, np.sin(a0), np.cos(a0)).astype(np.float32)
    q0 = np.where(even, np.cos(a0), -np.sin(a0)).astype(np.float32)
    nblk = seq_len // br
    off = (br * np.arange(nblk, dtype=np.float64).reshape(-1, 1)) * w
    cbx = np.repeat(np.cos(off), _O, axis=0).astype(np.float32)
    sbx = np.repeat(np.sin(off), _O, axis=0).astype(np.float32)
    return (np.ascontiguousarray(p0.T), np.ascontiguousarray(q0.T),
            cbx, sbx)


_P0T, _Q0T, _CBX, _SBX = _pos_tables(_L, _E, _BR)


def _sc_gather(idx, table, start, num):
    info = plsc.get_sparse_core_info()
    nw = info.num_cores * info.num_subcores
    d = table.shape[1]
    b_per_w = num // nw
    ch = 128
    n_ch = b_per_w // ch

    mesh = plsc.VectorSubcoreMesh(core_axis_name="c", subcore_axis_name="s")

    @functools.partial(
        pl.kernel,
        mesh=mesh,
        out_type=jax.ShapeDtypeStruct((num, d), jnp.float32),
        scratch_types=[
            pltpu.VMEM((b_per_w,), jnp.int32),
            pltpu.VMEM((b_per_w, d), jnp.float32),
            pltpu.SemaphoreType.DMA,
            pltpu.SemaphoreType.DMA,
        ],
    )
    def gather_kernel(idx_hbm, table_hbm, out_hbm, idx_v, rows_v, sem,
                      wsem):
        wid = lax.axis_index("s") * info.num_cores + lax.axis_index("c")
        base = wid * b_per_w
        pltpu.sync_copy(idx_hbm.at[pl.ds(start + base, b_per_w)], idx_v)
        copies = [
            pltpu.async_copy(
                table_hbm.at[idx_v.at[pl.ds(j * ch, ch)]],
                rows_v.at[pl.ds(j * ch, ch)],
                sem,
            )
            for j in range(n_ch)
        ]
        writes = []
        for j, c in enumerate(copies):
            c.wait()
            writes.append(pltpu.async_copy(
                rows_v.at[pl.ds(j * ch, ch)],
                out_hbm.at[pl.ds(base + j * ch, ch)],
                wsem,
            ))
        for w in writes:
            w.wait()

    return gather_kernel(idx, table)


def _tc_pos_proj(wq, bq):
    o, e = wq.shape
    br = _BR
    nblk = _NBLK

    def body(p0t_ref, q0t_ref, cbx_ref, sbx_ref, wq_ref, bq_ref, out_ref):
        wqt = jnp.concatenate([wq_ref[...]] * nblk, axis=0)
        bqt = jnp.concatenate([bq_ref[...]] * nblk, axis=0)
        a = lax.dot_general(
            wqt * cbx_ref[...], p0t_ref[...], (((1,), (0,)), ((), ())),
            preferred_element_type=jnp.float32)
        b = lax.dot_general(
            wqt * sbx_ref[...], q0t_ref[...], (((1,), (0,)), ((), ())),
            preferred_element_type=jnp.float32)
        out_ref[...] = (a + b + bqt).reshape(nblk, o, br)

    return pl.pallas_call(
        body,
        in_specs=[
            pl.BlockSpec((e, br), lambda: (0, 0)),
            pl.BlockSpec((e, br), lambda: (0, 0)),
            pl.BlockSpec((nblk * o, e), lambda: (0, 0)),
            pl.BlockSpec((nblk * o, e), lambda: (0, 0)),
            pl.BlockSpec((o, e), lambda: (0, 0)),
            pl.BlockSpec((o, 1), lambda: (0, 0)),
        ],
        out_specs=pl.BlockSpec((nblk, o, br), lambda: (0, 0, 0)),
        out_shape=jax.ShapeDtypeStruct((nblk, o, br), jnp.float32),
    )(jnp.asarray(_P0T), jnp.asarray(_Q0T), jnp.asarray(_CBX),
      jnp.asarray(_SBX), wq, bq.reshape(o, 1))


def _tc_project(emb, wq, pp, blk0, carry=None):
    o = wq.shape[0]
    e = wq.shape[1]
    br = _BR
    nb = emb.shape[0] // br

    def body(*refs):
        if carry is None:
            emba_ref, embb_ref, wq_ref, ppa_ref, ppb_ref, out_ref = refs
        else:
            _, emba_ref, embb_ref, wq_ref, ppa_ref, ppb_ref, out_ref = refs
        wq2 = wq_ref[...]
        z = jnp.zeros_like(wq2)
        c0 = jnp.concatenate([wq2, z], axis=1)
        c1 = jnp.concatenate([z, wq2], axis=1)
        w2 = jnp.concatenate(
            [c0.reshape(o, 1, 2 * e), c1.reshape(o, 1, 2 * e)], axis=1
        ).reshape(2 * o, 2 * e)
        emb2 = jnp.concatenate([emba_ref[...], embb_ref[...]], axis=1)
        acc = lax.dot_general(
            w2, emb2, (((1,), (1,)), ((), ())),
            preferred_element_type=jnp.float32)
        pp2 = jnp.concatenate(
            [ppa_ref[...].reshape(o, 1, br), ppb_ref[...].reshape(o, 1, br)],
            axis=1).reshape(o, 2 * br)
        out_ref[...] = acc.reshape(o, 2 * br) + pp2

    in_specs = [
        pl.BlockSpec((br, e), lambda j: (2 * j, 0)),
        pl.BlockSpec((br, e), lambda j: (2 * j + 1, 0)),
        pl.BlockSpec((o, e), lambda j: (0, 0)),
        pl.BlockSpec((1, o, br), lambda j: (blk0 + 2 * j, 0, 0)),
        pl.BlockSpec((1, o, br), lambda j: (blk0 + 2 * j + 1, 0, 0)),
    ]
    args = [emb, emb, wq, pp, pp]
    kwargs = {}
    if carry is not None:
        in_specs = [pl.BlockSpec(memory_space=pl.ANY)] + in_specs
        args = [carry] + args
        kwargs["input_output_aliases"] = {0: 0}

    return pl.pallas_call(
        body,
        grid=(nb // 2,),
        in_specs=in_specs,
        out_specs=pl.BlockSpec((o, 2 * br), lambda j: (0, blk0 // 2 + j)),
        out_shape=jax.ShapeDtypeStruct((o, _L), jnp.float32),
        **kwargs,
    )(*args)


def kernel(inputs, table, Wq, bq):
    pp = _tc_pos_proj(Wq, bq)
    l = inputs.shape[0]
    emb = _sc_gather(inputs, table, 0, l)
    out_t = _tc_project(emb, Wq, pp, blk0=0)
    return out_t.T
